# baseline (speedup 1.0000x reference)
import jax
import jax.numpy as jnp
from jax.experimental import pallas as pl
from jax.experimental.pallas import tpu as pltpu
from jax.experimental.pallas import tpu_sc as plsc

_WINDOW = 128
_TC_BLK = 4096
_SC_ROWS = 65536


def _transform_body(e_ref, w_ref, t_ref):
    h = jax.lax.dot_general(
        e_ref[...], w_ref[...],
        (((1,), (1,)), ((), ())),
        preferred_element_type=jnp.float32,
    )
    t_ref[...] = h * jax.nn.sigmoid(h)


def _onehot_body(idx_ref, t_ref, o_ref):
    v = t_ref.shape[0]
    blk = idx_ref.shape[1]
    iota = jax.lax.broadcasted_iota(jnp.int32, (v, blk), 0)
    oht = (jnp.broadcast_to(idx_ref[...], (v, blk)) == iota).astype(jnp.float32)
    o_ref[...] = jax.lax.dot_general(
        oht, t_ref[...],
        (((0,), (0,)), ((), ())),
        preferred_element_type=jnp.float32,
    )


def kernel(atom_type, embedding_table, W):
    n_atoms = atom_type.shape[0]
    v, d = embedding_table.shape

    v_pad = -(-v // 8) * 8
    e = jnp.pad(embedding_table, ((0, v_pad - v), (0, 0)))
    table = pl.pallas_call(
        _transform_body,
        out_shape=jax.ShapeDtypeStruct((v_pad, d), jnp.float32),
    )(e, W)

    sc_windows = _SC_ROWS // _WINDOW
    idx = atom_type.reshape(1, n_atoms).astype(jnp.int32)
    blk0 = _SC_ROWS // _TC_BLK
    n_tc = n_atoms - _SC_ROWS
    grid_tc = -(-n_tc // _TC_BLK)

    tc_part = pl.pallas_call(
        _onehot_body,
        grid=(grid_tc,),
        in_specs=[
            pl.BlockSpec((1, _TC_BLK), lambda i: (0, blk0 + i)),
            pl.BlockSpec((v_pad, d), lambda i: (0, 0)),
        ],
        out_specs=pl.BlockSpec((_TC_BLK, d), lambda i: (i, 0)),
        out_shape=jax.ShapeDtypeStruct((n_tc, d), jnp.float32),
    )(idx, table)

    mesh = plsc.VectorSubcoreMesh(
        core_axis_name="core", subcore_axis_name="subcore"
    )

    @pl.kernel(
        out_type=jax.ShapeDtypeStruct((n_atoms, d), jnp.float32),
        mesh=mesh,
        scratch_types=[pltpu.VMEM_SHARED((v_pad, d), jnp.float32)],
    )
    def gather_kernel(t_hbm, i_hbm, o_hbm, t_shared):
        @pl.when(jax.lax.axis_index("subcore") == 0)
        def _load_table():
            pltpu.sync_copy(t_hbm, t_shared)

        plsc.subcore_barrier()

        def body(i_vmem, o_vmem):
            pltpu.sync_copy(t_shared.at[i_vmem.at[0]], o_vmem)

        pltpu.emit_pipeline(
            body,
            grid=(sc_windows,),
            in_specs=[pl.BlockSpec((1, _WINDOW), index_map=lambda i: (0, i))],
            out_specs=[pl.BlockSpec((_WINDOW, d), index_map=lambda i: (i, 0))],
            core_axis_name=("core", "subcore"),
            dimension_semantics=(pltpu.PARALLEL,),
        )(i_hbm, o_hbm)

    sc_full = gather_kernel(table, idx)

    out = jax.lax.dynamic_update_slice(sc_full, tc_part, (_SC_ROWS, 0))
    return out.reshape(n_atoms, 1, d)


# speedup vs baseline: 1.1695x; 1.1695x over previous
import jax
import jax.numpy as jnp
from jax.experimental import pallas as pl
from jax.experimental.pallas import tpu as pltpu
from jax.experimental.pallas import tpu_sc as plsc


def _transform_body(e_ref, w_ref, t_ref):
    h = jax.lax.dot_general(
        e_ref[...], w_ref[...],
        (((1,), (1,)), ((), ())),
        preferred_element_type=jnp.float32,
    )
    t_ref[...] = h * jax.nn.sigmoid(h)


def kernel(atom_type, embedding_table, W):
    n_atoms = atom_type.shape[0]
    v, d = embedding_table.shape

    v_pad = -(-v // 8) * 8
    e = jnp.pad(embedding_table, ((0, v_pad - v), (0, 0)))
    table = pl.pallas_call(
        _transform_body,
        out_shape=jax.ShapeDtypeStruct((v_pad, d), jnp.float32),
    )(e, W)

    window = 128
    grid = n_atoms // window
    n_tail = n_atoms - grid * window
    tail_base = grid * window

    idx = atom_type.reshape(1, n_atoms).astype(jnp.int32)
    mesh = plsc.VectorSubcoreMesh(
        core_axis_name="core", subcore_axis_name="subcore"
    )

    @pl.kernel(
        out_type=jax.ShapeDtypeStruct((n_atoms, d), jnp.float32),
        mesh=mesh,
        scratch_types=[
            pltpu.VMEM_SHARED((v_pad, d), jnp.float32),
            pltpu.VMEM((n_tail,), jnp.int32),
            pltpu.VMEM((n_tail, d), jnp.float32),
        ],
    )
    def gather_kernel(t_hbm, i_hbm, o_hbm, t_shared, tail_idx, tail_rows):
        @pl.when(jax.lax.axis_index("subcore") == 0)
        def _load_table():
            pltpu.sync_copy(t_hbm, t_shared)

        plsc.subcore_barrier()

        def body(i_vmem, o_vmem):
            pltpu.sync_copy(t_shared.at[i_vmem.at[0]], o_vmem)

        pltpu.emit_pipeline(
            body,
            grid=(grid,),
            in_specs=[pl.BlockSpec((1, window), index_map=lambda i: (0, i))],
            out_specs=[pl.BlockSpec((window, d), index_map=lambda i: (i, 0))],
            core_axis_name=("core", "subcore"),
            dimension_semantics=(pltpu.PARALLEL,),
        )(i_hbm, o_hbm)

        wid = (jax.lax.axis_index("subcore") * 2 + jax.lax.axis_index("core"))

        @pl.when(wid == 0)
        def _tail():
            pltpu.sync_copy(i_hbm.at[0, pl.ds(tail_base, n_tail)], tail_idx)
            pltpu.sync_copy(t_shared.at[tail_idx], tail_rows)
            pltpu.sync_copy(tail_rows, o_hbm.at[pl.ds(tail_base, n_tail)])

    out = gather_kernel(table, idx)
    return out.reshape(n_atoms, 1, d)


# speedup vs baseline: 1.2547x; 1.0728x over previous
import jax
import jax.numpy as jnp
from jax.experimental import pallas as pl
from jax.experimental.pallas import tpu as pltpu
from jax.experimental.pallas import tpu_sc as plsc


def _transform_body(e_ref, w_ref, t_ref):
    h = jax.lax.dot_general(
        e_ref[...], w_ref[...],
        (((1,), (1,)), ((), ())),
        preferred_element_type=jnp.float32,
    )
    t_ref[...] = h * jax.nn.sigmoid(h)


def kernel(atom_type, embedding_table, W):
    n_atoms = atom_type.shape[0]
    v, d = embedding_table.shape

    v_pad = -(-v // 8) * 8
    e = jnp.pad(embedding_table, ((0, v_pad - v), (0, 0)))
    table = pl.pallas_call(
        _transform_body,
        out_shape=jax.ShapeDtypeStruct((v_pad, d), jnp.float32),
    )(e, W)

    window = 128
    grid = n_atoms // window
    n_tail = n_atoms - grid * window
    tail_base = grid * window

    idx = atom_type.reshape(1, n_atoms).astype(jnp.int32)
    mesh = plsc.VectorSubcoreMesh(
        core_axis_name="core", subcore_axis_name="subcore"
    )

    @pl.kernel(
        out_type=jax.ShapeDtypeStruct((n_atoms, d), jnp.float32),
        mesh=mesh,
        scratch_types=[
            pltpu.VMEM_SHARED((v_pad, d), jnp.float32),
            pltpu.VMEM((n_tail,), jnp.int32),
            pltpu.VMEM((n_tail, d), jnp.float32),
        ],
    )
    def gather_kernel(t_hbm, i_hbm, o_hbm, t_shared, tail_idx, tail_rows):
        @pl.when(jax.lax.axis_index("subcore") == 0)
        def _load_table():
            pltpu.sync_copy(t_hbm, t_shared)

        plsc.subcore_barrier()

        def body(i_vmem, o_vmem):
            pltpu.sync_copy(t_shared.at[i_vmem.at[0]], o_vmem)

        split = 512

        pltpu.emit_pipeline(
            body,
            grid=(split,),
            in_specs=[pl.BlockSpec((1, window), index_map=lambda i: (0, i))],
            out_specs=[pl.BlockSpec((window, d), index_map=lambda i: (i, 0))],
            core_axis_name=("core", "subcore"),
            dimension_semantics=(pltpu.PARALLEL,),
        )(i_hbm, o_hbm)

        pltpu.emit_pipeline(
            body,
            grid=(grid - split,),
            in_specs=[pl.BlockSpec((1, window),
                                   index_map=lambda i: (0, split + i))],
            out_specs=[pl.BlockSpec((window, d),
                                    index_map=lambda i: (split + i, 0))],
            core_axis_name=("core", "subcore"),
            dimension_semantics=(pltpu.PARALLEL,),
        )(i_hbm, o_hbm)

        wid = (jax.lax.axis_index("subcore") * 2 + jax.lax.axis_index("core"))

        @pl.when(wid == 0)
        def _tail():
            pltpu.sync_copy(i_hbm.at[0, pl.ds(tail_base, n_tail)], tail_idx)
            pltpu.sync_copy(t_shared.at[tail_idx], tail_rows)
            pltpu.sync_copy(tail_rows, o_hbm.at[pl.ds(tail_base, n_tail)])

    out = gather_kernel(table, idx)
    return out.reshape(n_atoms, 1, d)
